# Initial kernel scaffold; baseline (speedup 1.0000x reference)
import jax
import jax.numpy as jnp
from jax.experimental import pallas as pl


def kernel(x, edge_label_index, W, b):
    raise NotImplementedError("write your pallas kernel here")


# speedup vs baseline: 13.8875x; 13.8875x over previous
import functools

import jax
import jax.numpy as jnp
from jax import lax
from jax.experimental import pallas as pl
from jax.experimental.pallas import tpu as pltpu
from jax.experimental.pallas import tpu_sc as plsc

_N_NODES = 10000
_D = 128
_N_EDGES = 320000
_N_CORES = 2
_N_SUBCORES = 16
_N_WORKERS = _N_CORES * _N_SUBCORES
_E_PER_W = _N_EDGES // _N_WORKERS
_L = 16


def _scores_body(x_ref, wt_ref, hb_ref, out_ref):
    out_ref[...] = (
        jnp.dot(x_ref[...], wt_ref[...], preferred_element_type=jnp.float32)
        + hb_ref[0, 0]
    )


_scores_call = pl.pallas_call(
    _scores_body,
    out_shape=jax.ShapeDtypeStruct((_N_NODES, 2), jnp.float32),
)


_sc_mesh = plsc.VectorSubcoreMesh(core_axis_name="c", subcore_axis_name="s")


@functools.partial(
    pl.kernel,
    out_type=jax.ShapeDtypeStruct((_N_EDGES,), jnp.float32),
    mesh=_sc_mesh,
    scratch_types=[
        pltpu.VMEM((_E_PER_W,), jnp.int32),
        pltpu.VMEM((_E_PER_W,), jnp.int32),
        pltpu.VMEM((_E_PER_W,), jnp.float32),
        pltpu.VMEM((_E_PER_W,), jnp.float32),
        pltpu.VMEM((_E_PER_W,), jnp.float32),
        pltpu.SemaphoreType.DMA,
    ],
)
def _edge_call(
    p_hbm, q_hbm, src_hbm, dst_hbm, out_hbm,
    src_v, dst_v, p_v, q_v, out_v, sem,
):
    wid = lax.axis_index("s") * _N_CORES + lax.axis_index("c")
    base = wid * _E_PER_W
    pltpu.sync_copy(src_hbm.at[pl.ds(base, _E_PER_W)], src_v)
    pltpu.sync_copy(dst_hbm.at[pl.ds(base, _E_PER_W)], dst_v)
    cp = pltpu.make_async_copy(p_hbm.at[src_v], p_v, sem)
    cq = pltpu.make_async_copy(q_hbm.at[dst_v], q_v, sem)
    cp.start()
    cq.start()
    cp.wait()
    cq.wait()

    def body(i, carry):
        off = i * _L
        out_v[pl.ds(off, _L)] = p_v[pl.ds(off, _L)] + q_v[pl.ds(off, _L)]
        return carry

    lax.fori_loop(0, _E_PER_W // _L, body, 0)
    pltpu.sync_copy(out_v, out_hbm.at[pl.ds(base, _E_PER_W)])


def kernel(x, edge_label_index, W, b):
    x = x.astype(jnp.float32)
    eli = edge_label_index.astype(jnp.int32)
    wt = jnp.stack([W[0, :_D], W[0, _D:]], axis=1)
    hb = (b * 0.5).reshape(1, 1).astype(jnp.float32)
    scores = _scores_call(x, wt, hb)
    out = _edge_call(scores[:, 0], scores[:, 1], eli[0], eli[1])
    return out.reshape(_N_EDGES, 1)


# speedup vs baseline: 27.3914x; 1.9724x over previous
import functools

import jax
import jax.numpy as jnp
from jax import lax
from jax.experimental import pallas as pl
from jax.experimental.pallas import tpu as pltpu
from jax.experimental.pallas import tpu_sc as plsc

_N_NODES = 10000
_D = 128
_N_EDGES = 320000
_N_CORES = 2
_N_SUBCORES = 16
_N_WORKERS = _N_CORES * _N_SUBCORES
_E_PER_W = _N_EDGES // _N_WORKERS
_L = 16


def _scores_body(x_ref, wt_ref, hb_ref, out_ref):
    out_ref[...] = (
        jnp.dot(x_ref[...], wt_ref[...], preferred_element_type=jnp.float32)
        + hb_ref[0, 0]
    )


_scores_call = pl.pallas_call(
    _scores_body,
    out_shape=jax.ShapeDtypeStruct((_N_NODES, 2), jnp.float32),
)


_sc_mesh = plsc.VectorSubcoreMesh(core_axis_name="c", subcore_axis_name="s")


@functools.partial(
    pl.kernel,
    out_type=jax.ShapeDtypeStruct((_N_EDGES,), jnp.float32),
    mesh=_sc_mesh,
    compiler_params=pltpu.CompilerParams(needs_layout_passes=False),
    scratch_types=[
        pltpu.VMEM((_N_NODES,), jnp.float32),
        pltpu.VMEM((_N_NODES,), jnp.float32),
        pltpu.VMEM((_E_PER_W,), jnp.int32),
        pltpu.VMEM((_E_PER_W,), jnp.int32),
        pltpu.VMEM((_E_PER_W,), jnp.float32),
    ],
)
def _edge_call(
    p_hbm, q_hbm, src_hbm, dst_hbm, out_hbm,
    p_t, q_t, src_v, dst_v, out_v,
):
    wid = lax.axis_index("s") * _N_CORES + lax.axis_index("c")
    base = wid * _E_PER_W
    pltpu.sync_copy(p_hbm, p_t)
    pltpu.sync_copy(q_hbm, q_t)
    pltpu.sync_copy(src_hbm.at[pl.ds(base, _E_PER_W)], src_v)
    pltpu.sync_copy(dst_hbm.at[pl.ds(base, _E_PER_W)], dst_v)

    def body(i, carry):
        off = i * _L
        pv = plsc.load_gather(p_t, [src_v[pl.ds(off, _L)]])
        qv = plsc.load_gather(q_t, [dst_v[pl.ds(off, _L)]])
        out_v[pl.ds(off, _L)] = pv + qv
        return carry

    lax.fori_loop(0, _E_PER_W // _L, body, 0)
    pltpu.sync_copy(out_v, out_hbm.at[pl.ds(base, _E_PER_W)])


def kernel(x, edge_label_index, W, b):
    x = x.astype(jnp.float32)
    eli = edge_label_index.astype(jnp.int32)
    wt = jnp.stack([W[0, :_D], W[0, _D:]], axis=1)
    hb = (b * 0.5).reshape(1, 1).astype(jnp.float32)
    scores = _scores_call(x, wt, hb)
    out = _edge_call(scores[:, 0], scores[:, 1], eli[0], eli[1])
    return out.reshape(_N_EDGES, 1)


# speedup vs baseline: 39.2378x; 1.4325x over previous
import functools

import jax
import jax.numpy as jnp
from jax import lax
from jax.experimental import pallas as pl
from jax.experimental.pallas import tpu as pltpu
from jax.experimental.pallas import tpu_sc as plsc

_N_NODES = 10000
_D = 128
_N_EDGES = 320000
_N_CORES = 2
_N_SUBCORES = 16
_N_WORKERS = _N_CORES * _N_SUBCORES
_E_PER_W = _N_EDGES // _N_WORKERS
_L = 16


def _scores_body(x_ref, wt_ref, hb_ref, out_ref):
    out_ref[...] = (
        lax.dot_general(
            wt_ref[...],
            x_ref[...],
            (((1,), (1,)), ((), ())),
            preferred_element_type=jnp.float32,
        )
        + hb_ref[0, 0]
    )


_scores_call = pl.pallas_call(
    _scores_body,
    out_shape=jax.ShapeDtypeStruct((2, _N_NODES), jnp.float32),
)


_sc_mesh = plsc.VectorSubcoreMesh(core_axis_name="c", subcore_axis_name="s")


@functools.partial(
    pl.kernel,
    out_type=jax.ShapeDtypeStruct((_N_EDGES,), jnp.float32),
    mesh=_sc_mesh,
    compiler_params=pltpu.CompilerParams(needs_layout_passes=False),
    scratch_types=[
        pltpu.VMEM((_N_NODES,), jnp.float32),
        pltpu.VMEM((_N_NODES,), jnp.float32),
        pltpu.VMEM((_E_PER_W,), jnp.int32),
        pltpu.VMEM((_E_PER_W,), jnp.int32),
        pltpu.VMEM((_E_PER_W,), jnp.float32),
    ],
)
def _edge_call(
    scores_hbm, eli_hbm, out_hbm,
    p_t, q_t, src_v, dst_v, out_v,
):
    wid = lax.axis_index("s") * _N_CORES + lax.axis_index("c")
    base = wid * _E_PER_W
    pltpu.sync_copy(scores_hbm.at[pl.ds(0, _N_NODES)], p_t)
    pltpu.sync_copy(scores_hbm.at[pl.ds(_N_NODES, _N_NODES)], q_t)
    pltpu.sync_copy(eli_hbm.at[pl.ds(base, _E_PER_W)], src_v)
    pltpu.sync_copy(eli_hbm.at[pl.ds(_N_EDGES + base, _E_PER_W)], dst_v)

    def body(i, carry):
        off = i * _L
        pv = plsc.load_gather(p_t, [src_v[pl.ds(off, _L)]])
        qv = plsc.load_gather(q_t, [dst_v[pl.ds(off, _L)]])
        out_v[pl.ds(off, _L)] = pv + qv
        return carry

    lax.fori_loop(0, _E_PER_W // _L, body, 0)
    pltpu.sync_copy(out_v, out_hbm.at[pl.ds(base, _E_PER_W)])


def kernel(x, edge_label_index, W, b):
    x = x.astype(jnp.float32)
    eli = edge_label_index.astype(jnp.int32)
    wt = W.reshape(2, _D)
    hb = (b * 0.5).reshape(1, 1).astype(jnp.float32)
    scores = _scores_call(x, wt, hb)
    out = _edge_call(scores.reshape(-1), eli.reshape(-1))
    return out.reshape(_N_EDGES, 1)
